# baseline (speedup 1.0000x reference)
import functools

import jax
import jax.numpy as jnp
from jax import lax
from jax.experimental import pallas as pl
from jax.experimental.pallas import tpu as pltpu
from jax.experimental.pallas import tpu_sc as plsc

D = 768

NUM_CORES = 2
NUM_SUBCORES = 16
NW = NUM_CORES * NUM_SUBCORES

SEQ_PAD = 80
MAX_IDX_CHUNK_BLKS = 32

TC_G = 16
K_CHUNKS = 4


def _sc_gather_body(ids_hbm, tok_tab, out_hbm, idx_v, rows_v, sem_g, sem_o,
                    *, blocks_per_w, idx_chunk_blks):
    wid = lax.axis_index("s") * NUM_CORES + lax.axis_index("c")
    base_row = wid * blocks_per_w * SEQ_PAD
    n_chunks = blocks_per_w // idx_chunk_blks

    def gather(i, buf):
        return pltpu.make_async_copy(
            tok_tab.at[idx_v.at[pl.ds(i * SEQ_PAD, SEQ_PAD)]],
            rows_v.at[buf], sem_g)

    def writeout(base, i, buf):
        return pltpu.make_async_copy(
            rows_v.at[buf], out_hbm.at[pl.ds(base + i * SEQ_PAD, SEQ_PAD)],
            sem_o)

    def chunk_body(c, carry):
        chunk_row = base_row + c * idx_chunk_blks * SEQ_PAD
        pltpu.sync_copy(ids_hbm.at[pl.ds(chunk_row, idx_chunk_blks * SEQ_PAD)],
                        idx_v)
        gather(0, 0).start()

        def body(i, carry2):
            buf = lax.rem(i, 2)
            nbuf = lax.rem(i + 1, 2)

            @pl.when(i >= 1)
            def _():
                writeout(chunk_row, i - 1, nbuf).wait()

            @pl.when(i + 1 < idx_chunk_blks)
            def _():
                gather(i + 1, nbuf).start()

            gather(i, buf).wait()
            writeout(chunk_row, i, buf).start()
            return carry2

        lax.fori_loop(0, idx_chunk_blks, body, 0)
        writeout(chunk_row, idx_chunk_blks - 1,
                 lax.rem(idx_chunk_blks - 1, 2)).wait()
        return carry

    lax.fori_loop(0, n_chunks, chunk_body, 0)


def _sc_gather(ids_pad_flat, token_table, n_rows):
    blocks_per_w = (n_rows // SEQ_PAD) // NW
    idx_chunk_blks = min(MAX_IDX_CHUNK_BLKS, blocks_per_w)
    assert blocks_per_w % idx_chunk_blks == 0
    mesh = plsc.VectorSubcoreMesh(core_axis_name="c", subcore_axis_name="s")
    run = pl.kernel(
        functools.partial(_sc_gather_body, blocks_per_w=blocks_per_w,
                          idx_chunk_blks=idx_chunk_blks),
        mesh=mesh,
        out_type=jax.ShapeDtypeStruct((n_rows, D), jnp.float32),
        scratch_types=[
            pltpu.VMEM((idx_chunk_blks * SEQ_PAD,), jnp.int32),
            pltpu.VMEM((2, SEQ_PAD, D), jnp.float32),
            pltpu.SemaphoreType.DMA,
            pltpu.SemaphoreType.DMA,
        ],
    )
    return run(ids_pad_flat, token_table)


def _tc_body(*refs, seq):
    rows_ref, pid_ref, ptab_ref, out_ref = refs[-4:]
    g = out_ref.shape[0]
    onehot = (pid_ref[...]
              == lax.broadcasted_iota(jnp.int32, (g * SEQ_PAD, seq), 1)
              ).astype(jnp.float32)
    pos = jax.lax.dot(onehot, ptab_ref[...],
                      preferred_element_type=jnp.float32)
    total = rows_ref[...] + pos
    for b in range(g):
        out_ref[b] = total[b * SEQ_PAD:b * SEQ_PAD + seq, :]


def _tc_addpos_chunk(prev_out, tok_rows, pid_pad_flat, position_table,
                     bsz, seq, chunk_b, chunk_off):
    grid = (chunk_b // TC_G,)
    step_off = chunk_off // TC_G
    data_specs = [
        pl.BlockSpec((TC_G * SEQ_PAD, D), lambda i: (i, 0)),
        pl.BlockSpec((TC_G * SEQ_PAD, 1), lambda i: (i, 0)),
        pl.BlockSpec((seq, D), lambda i: (0, 0)),
    ]
    if prev_out is None:
        in_specs, aliases, args = data_specs, {}, ()
    else:
        in_specs = ([pl.BlockSpec(memory_space=pltpu.MemorySpace.HBM)]
                    + data_specs)
        aliases, args = {0: 0}, (prev_out,)
    return pl.pallas_call(
        functools.partial(_tc_body, seq=seq),
        grid=grid,
        in_specs=in_specs,
        out_specs=pl.BlockSpec((TC_G, seq, D),
                               lambda i: (step_off + i, 0, 0)),
        out_shape=jax.ShapeDtypeStruct((bsz, seq, D), jnp.float32),
        input_output_aliases=aliases,
    )(*args, tok_rows, pid_pad_flat, position_table)


def kernel(input_ids, position_ids, token_table, position_table):
    bsz, seq = input_ids.shape
    assert seq <= SEQ_PAD and bsz % (K_CHUNKS * NW * TC_G) == 0
    pad = ((0, 0), (0, SEQ_PAD - seq))
    vocab = token_table.shape[0]
    spread = (jnp.arange(bsz, dtype=jnp.int32)[:, None] * (SEQ_PAD - seq)
              + jnp.arange(SEQ_PAD - seq, dtype=jnp.int32)[None, :]) % vocab
    ids_pad = jnp.concatenate(
        [input_ids.astype(jnp.int32), spread], axis=1).reshape(bsz * SEQ_PAD)
    pid_pad = jnp.pad(position_ids.astype(jnp.int32),
                      pad).reshape(bsz * SEQ_PAD, 1)

    chunk_b = bsz // K_CHUNKS
    chunk_rows = chunk_b * SEQ_PAD
    tok_chunks = [
        _sc_gather(lax.dynamic_slice(ids_pad, (k * chunk_rows,),
                                     (chunk_rows,)),
                   token_table, chunk_rows)
        for k in range(K_CHUNKS)
    ]

    out = None
    for k in range(K_CHUNKS):
        pid_k = lax.dynamic_slice(pid_pad, (k * chunk_rows, 0),
                                  (chunk_rows, 1))
        out = _tc_addpos_chunk(out, tok_chunks[k], pid_k, position_table,
                               bsz, seq, chunk_b, k * chunk_b)
    return out


# speedup vs baseline: 1.1413x; 1.1413x over previous
import functools

import jax
import jax.numpy as jnp
from jax import lax
from jax.experimental import pallas as pl
from jax.experimental.pallas import tpu as pltpu
from jax.experimental.pallas import tpu_sc as plsc

D = 768
LANES = 16
VREGS_PER_ROW = D // LANES

NUM_CORES = 2
NUM_SUBCORES = 16
NW = NUM_CORES * NUM_SUBCORES

W = 32
WINS_PER_CHUNK = 28


def _emb_body(ids_hbm, pid_hbm, tok_tab, pos_tab, out_hbm,
              idx_v, pidx_v, tok_v, pos_v, sidx_v, sem_t, sem_p, sem_o,
              *, bsz, seq, toks_per_w):
    wid = lax.axis_index("s") * NUM_CORES + lax.axis_index("c")
    tok0 = wid * toks_per_w
    chunk_toks = WINS_PER_CHUNK * W
    n_chunks = toks_per_w // chunk_toks

    def gathers(i, buf):
        return (
            pltpu.make_async_copy(
                tok_tab.at[idx_v.at[pl.ds(i * W, W)]], tok_v.at[buf], sem_t),
            pltpu.make_async_copy(
                pos_tab.at[pidx_v.at[pl.ds(i * W, W)]], pos_v.at[buf], sem_p),
        )

    def scatter(buf):
        return pltpu.make_async_copy(
            tok_v.at[buf], out_hbm.at[sidx_v.at[buf]], sem_o)

    def chunk_body(c, carry):
        chunk_tok = tok0 + c * chunk_toks
        pltpu.sync_copy(ids_hbm.at[pl.ds(chunk_tok, chunk_toks)], idx_v)
        pltpu.sync_copy(pid_hbm.at[pl.ds(chunk_tok, chunk_toks)], pidx_v)
        for g in gathers(0, 0):
            g.start()

        def body(i, carry2):
            buf = lax.rem(i, 2)
            nbuf = lax.rem(i + 1, 2)

            @pl.when(i >= 1)
            def _():
                scatter(nbuf).wait()

            @pl.when(i + 1 < WINS_PER_CHUNK)
            def _():
                for g in gathers(i + 1, nbuf):
                    g.start()

            gt, gp = gathers(i, buf)
            gt.wait()
            gp.wait()

            def add_row(r, c2):
                for cc in range(VREGS_PER_ROW):
                    sl = pl.ds(cc * LANES, LANES)
                    tok_v[buf, r, sl] = tok_v[buf, r, sl] + pos_v[buf, r, sl]
                return c2

            lax.fori_loop(0, W, add_row, 0)

            win_tok = chunk_tok + i * W
            for k in range(W // LANES):
                t = lax.broadcasted_iota(jnp.int32, (LANES,), 0) + (
                    win_tok + k * LANES)
                s = lax.rem(t, seq)
                b = lax.div(t, seq)
                sidx_v[buf, pl.ds(k * LANES, LANES)] = s * bsz + b

            scatter(buf).start()
            return carry2

        lax.fori_loop(0, WINS_PER_CHUNK, body, 0)
        scatter(lax.rem(WINS_PER_CHUNK - 1, 2)).wait()
        return carry

    lax.fori_loop(0, n_chunks, chunk_body, 0)


def kernel(input_ids, position_ids, token_table, position_table):
    bsz, seq = input_ids.shape
    n = bsz * seq
    toks_per_w = n // NW
    assert toks_per_w * NW == n
    assert toks_per_w % (WINS_PER_CHUNK * W) == 0

    ids = input_ids.astype(jnp.int32).reshape(n)
    pid = position_ids.astype(jnp.int32).reshape(n)

    mesh = plsc.VectorSubcoreMesh(core_axis_name="c", subcore_axis_name="s")
    run = pl.kernel(
        functools.partial(_emb_body, bsz=bsz, seq=seq,
                          toks_per_w=toks_per_w),
        mesh=mesh,
        out_type=jax.ShapeDtypeStruct((n, D), jnp.float32),
        scratch_types=[
            pltpu.VMEM((WINS_PER_CHUNK * W,), jnp.int32),
            pltpu.VMEM((WINS_PER_CHUNK * W,), jnp.int32),
            pltpu.VMEM((2, W, D), jnp.float32),
            pltpu.VMEM((2, W, D), jnp.float32),
            pltpu.VMEM((2, W), jnp.int32),
            pltpu.SemaphoreType.DMA,
            pltpu.SemaphoreType.DMA,
            pltpu.SemaphoreType.DMA,
        ],
    )
    out_sm = run(ids, pid, token_table, position_table)
    return out_sm.reshape(seq, bsz, D).transpose(1, 0, 2)


# speedup vs baseline: 1.9443x; 1.7036x over previous
import functools

import jax
import jax.numpy as jnp
from jax import lax
from jax.experimental import pallas as pl
from jax.experimental.pallas import tpu as pltpu
from jax.experimental.pallas import tpu_sc as plsc

D = 768
LANES = 16
VREGS_PER_ROW = D // LANES

NUM_CORES = 2
NUM_SUBCORES = 16
NW = NUM_CORES * NUM_SUBCORES

W = 32
WINS_PER_CHUNK = 28


def _emb_body(ids_hbm, pid_hbm, tok_tab, pos_tab, out_hbm,
              idx_v, pidx_v, tok_v, pos_v, sidx_v, sem_t, sem_p, sem_o,
              *, bsz, seq, toks_per_w):
    wid = lax.axis_index("s") * NUM_CORES + lax.axis_index("c")
    tok0 = wid * toks_per_w
    chunk_toks = WINS_PER_CHUNK * W
    n_chunks = toks_per_w // chunk_toks

    def gathers(i, buf):
        return (
            pltpu.make_async_copy(
                tok_tab.at[idx_v.at[pl.ds(i * W, W)]], tok_v.at[buf], sem_t),
            pltpu.make_async_copy(
                pos_tab.at[pidx_v.at[pl.ds(i * W, W)]], pos_v.at[buf], sem_p),
        )

    def scatter(buf):
        return pltpu.make_async_copy(
            tok_v.at[buf], out_hbm.at[sidx_v.at[buf]], sem_o)

    def chunk_body(c, carry):
        chunk_tok = tok0 + c * chunk_toks
        pltpu.sync_copy(ids_hbm.at[pl.ds(chunk_tok, chunk_toks)], idx_v)
        pltpu.sync_copy(pid_hbm.at[pl.ds(chunk_tok, chunk_toks)], pidx_v)
        for g in gathers(0, 0):
            g.start()

        def body(i, carry2):
            buf = lax.rem(i, 2)
            nbuf = lax.rem(i + 1, 2)

            @pl.when(i >= 1)
            def _():
                scatter(nbuf).wait()

            @pl.when(i + 1 < WINS_PER_CHUNK)
            def _():
                for g in gathers(i + 1, nbuf):
                    g.start()

            gt, gp = gathers(i, buf)
            gt.wait()
            gp.wait()

            def add_row(r, c2):
                for cc in range(VREGS_PER_ROW):
                    sl = pl.ds(cc * LANES, LANES)
                    tok_v[buf, r, sl] = tok_v[buf, r, sl] + pos_v[buf, r, sl]
                return c2

            lax.fori_loop(0, W, add_row, 0)

            win_tok = chunk_tok + i * W
            for k in range(W // LANES):
                t = lax.broadcasted_iota(jnp.int32, (LANES,), 0) + (
                    win_tok + k * LANES)
                s = lax.rem(t, seq)
                b = lax.div(t, seq)
                sidx_v[buf, pl.ds(k * LANES, LANES)] = s * bsz + b

            scatter(buf).start()
            return carry2

        lax.fori_loop(0, WINS_PER_CHUNK, body, 0)
        scatter(lax.rem(WINS_PER_CHUNK - 1, 2)).wait()
        return carry

    lax.fori_loop(0, n_chunks, chunk_body, 0)


def kernel(input_ids, position_ids, token_table, position_table):
    bsz, seq = input_ids.shape
    n = bsz * seq
    toks_per_w = n // NW
    assert toks_per_w * NW == n
    assert toks_per_w % (WINS_PER_CHUNK * W) == 0

    ids = input_ids.astype(jnp.int32).reshape(n)
    pos_rep = jnp.tile(position_table, (NW, 1))
    pid = position_ids.astype(jnp.int32).reshape(n)
    pid = pid + (jnp.arange(n, dtype=jnp.int32) // toks_per_w) * seq

    mesh = plsc.VectorSubcoreMesh(core_axis_name="c", subcore_axis_name="s")
    run = pl.kernel(
        functools.partial(_emb_body, bsz=bsz, seq=seq,
                          toks_per_w=toks_per_w),
        mesh=mesh,
        out_type=jax.ShapeDtypeStruct((n, D), jnp.float32),
        scratch_types=[
            pltpu.VMEM((WINS_PER_CHUNK * W,), jnp.int32),
            pltpu.VMEM((WINS_PER_CHUNK * W,), jnp.int32),
            pltpu.VMEM((2, W, D), jnp.float32),
            pltpu.VMEM((2, W, D), jnp.float32),
            pltpu.VMEM((2, W), jnp.int32),
            pltpu.SemaphoreType.DMA,
            pltpu.SemaphoreType.DMA,
            pltpu.SemaphoreType.DMA,
        ],
    )
    out_sm = run(ids, pid, token_table, pos_rep)
    return out_sm.reshape(seq, bsz, D).transpose(1, 0, 2)
